# Initial kernel scaffold; baseline (speedup 1.0000x reference)
import jax
import jax.numpy as jnp
from jax.experimental import pallas as pl


def kernel(atom_types, elec_features, nuclear_table, W_elec, W_final, b_final):
    raise NotImplementedError("write your pallas kernel here")


# speedup vs baseline: 2.1296x; 2.1296x over previous
import functools

import jax
import jax.numpy as jnp
from jax import lax
from jax.experimental import pallas as pl
from jax.experimental.pallas import tpu as pltpu
from jax.experimental.pallas import tpu_sc as plsc

D = 128
TPAD = 128
N_ATOMS = 100000
NC, NS = 2, 16
NW = NC * NS
B_PAD = 100352
B_PER_W = B_PAD // NW
CHUNK = 392
NCHUNK = B_PER_W // CHUNK


def _table_body(nuc_ref, elec_ref, we_ref, wf_ref, b_ref, out_ref):
    combined = nuc_ref[...] + jnp.dot(
        elec_ref[...], we_ref[...], preferred_element_type=jnp.float32)
    h = jnp.dot(combined, wf_ref[...],
                preferred_element_type=jnp.float32) + b_ref[...]
    out_ref[...] = h * jax.nn.sigmoid(h)


def _fused_table(nuc_p, elec_p, we_p, wf, b2):
    return pl.pallas_call(
        _table_body,
        out_shape=jax.ShapeDtypeStruct((TPAD, D), jnp.float32),
    )(nuc_p, elec_p, we_p, wf, b2)


_MESH = plsc.VectorSubcoreMesh(core_axis_name="c", subcore_axis_name="s")


@functools.partial(
    pl.kernel,
    mesh=_MESH,
    out_type=jax.ShapeDtypeStruct((B_PAD, D), jnp.float32),
    scratch_types=[
        pltpu.VMEM((B_PER_W,), jnp.int32),
        pltpu.VMEM((CHUNK, D), jnp.float32),
        pltpu.SemaphoreType.DMA,
    ],
)
def _gather_rows(table_hbm, idx_hbm, out_hbm, idx_v, rows_v, sem):
    wid = lax.axis_index("s") * NC + lax.axis_index("c")
    base = wid * B_PER_W
    pltpu.sync_copy(idx_hbm.at[pl.ds(base, B_PER_W)], idx_v)
    for g in range(NCHUNK):
        pltpu.async_copy(
            table_hbm.at[idx_v.at[pl.ds(g * CHUNK, CHUNK)]], rows_v, sem
        ).wait()
        pltpu.sync_copy(rows_v, out_hbm.at[pl.ds(base + g * CHUNK, CHUNK)])


def kernel(atom_types, elec_features, nuclear_table, W_elec, W_final, b_final):
    elec_dim = elec_features.shape[1]
    nrows = nuclear_table.shape[0]
    nuc_p = jnp.zeros((TPAD, D), jnp.float32).at[:nrows].set(nuclear_table)
    elec_p = jnp.zeros((TPAD, D), jnp.float32).at[:nrows, :elec_dim].set(
        elec_features)
    we_p = jnp.zeros((D, D), jnp.float32).at[:elec_dim].set(W_elec)
    b2 = b_final.reshape(1, D)

    table = _fused_table(nuc_p, elec_p, we_p, W_final, b2)

    idx_p = jnp.pad(atom_types.astype(jnp.int32), (0, B_PAD - N_ATOMS))
    out_p = _gather_rows(table, idx_p)
    return out_p[:N_ATOMS]
